# baseline (speedup 1.0000x reference)
import functools

import jax
import jax.numpy as jnp
from jax import lax
from jax.experimental import pallas as pl
from jax.experimental.pallas import tpu as pltpu
from jax.experimental.pallas import tpu_sc as plsc

N_V = 10000
N_HE = 2000
N_INC = 320000
D = 128
INV_SQRT_D = 0.08838834764831845
LOG2E = 1.4426950408889634
QSCALE = INV_SQRT_D * LOG2E

N_V_PAD = 10240
N_HE_PAD = 2048
NCELLS = N_V_PAD * N_HE_PAD

NC = 2
NS = 16
N_CHUNKS = 16
CCH = NCELLS // N_CHUNKS
SHARE = CCH // NS
SC_CH = 128
N_INC_P = 327680
PER_SUB = N_INC_P // NS
SC_FULL = PER_SUB // SC_CH
NQ = SC_FULL // 4

BV = 512
BE = 256


def _sidx_body(node, hedge, s1_out, s2_out):
    lane = lax.broadcasted_iota(jnp.int32, node.shape, 1)
    n = node[...]
    h = hedge[...]
    f1 = n * N_HE_PAD + h
    f2 = h * N_V_PAD + n
    for k in range(N_CHUNKS):
        l1 = f1 - k * CCH
        s1_out[k] = jnp.where((l1 >= 0) & (l1 < CCH), l1, CCH + lane)
        l2 = f2 - k * CCH
        s2_out[k] = jnp.where((l2 >= 0) & (l2 < CCH), l2, CCH + lane)


def _make_sidx():
    rows = N_INC_P // D
    br = rows // 5
    return pl.pallas_call(
        _sidx_body,
        grid=(5,),
        in_specs=[
            pl.BlockSpec((br, D), lambda i: (i, 0)),
            pl.BlockSpec((br, D), lambda i: (i, 0)),
        ],
        out_specs=[
            pl.BlockSpec((N_CHUNKS, br, D), lambda i: (0, i, 0)),
            pl.BlockSpec((N_CHUNKS, br, D), lambda i: (0, i, 0)),
        ],
        out_shape=[
            jax.ShapeDtypeStruct((N_CHUNKS, rows, D), jnp.int32),
            jax.ShapeDtypeStruct((N_CHUNKS, rows, D), jnp.int32),
        ],
    )


def _count_body(s1_hbm, s2_hbm, zc_hbm, ones_hbm, c1_hbm, c2_hbm,
                ib0, ib1, ib2, ib3, ones_v, chunk,
                is0, is1, is2, is3, ss0, ss1, ss2, ss3):
    cid = lax.axis_index("c")
    sid = lax.axis_index("s")
    woff = sid * PER_SUB
    ibs = (ib0, ib1, ib2, ib3)
    isems = (is0, is1, is2, is3)
    ssems = (ss0, ss1, ss2, ss3)

    pltpu.sync_copy(ones_hbm, ones_v)

    for src_hbm, out_hbm in ((s1_hbm, c1_hbm), (s2_hbm, c2_hbm)):
        for m in range(N_CHUNKS // NC):
            k = cid * (N_CHUNKS // NC) + m
            base = k * CCH

            pltpu.sync_copy(zc_hbm.at[pl.ds(sid * SHARE, SHARE)],
                            chunk.at[pl.ds(sid * SHARE, SHARE)])
            plsc.subcore_barrier()

            def issue_idx(c, buf, sem):
                pltpu.async_copy(src_hbm.at[k, pl.ds(woff + c * SC_CH, SC_CH)],
                                 buf, sem)

            def wait_idx(c, buf, sem):
                pltpu.make_async_copy(
                    src_hbm.at[k, pl.ds(woff + c * SC_CH, SC_CH)],
                    buf, sem).wait()

            for j in range(4):
                issue_idx(j, ibs[j], isems[j])

            def quad(q, _):
                c0 = 4 * q
                descs = []
                for j in range(4):
                    wait_idx(c0 + j, ibs[j], isems[j])
                    descs.append(pltpu.async_copy(ones_v, chunk.at[ibs[j]],
                                                  ssems[j], add=True))
                for j in range(4):
                    descs[j].wait()

                    @pl.when(c0 + j + 4 < SC_FULL)
                    def _():
                        issue_idx(c0 + j + 4, ibs[j], isems[j])
                return 0

            lax.fori_loop(0, NQ, quad, 0)

            plsc.subcore_barrier()
            pltpu.sync_copy(chunk.at[pl.ds(sid * SHARE, SHARE)],
                            out_hbm.at[pl.ds(base + sid * SHARE, SHARE)])


_count_build = functools.partial(
    pl.kernel, _count_body,
    out_type=[
        jax.ShapeDtypeStruct((NCELLS,), jnp.float32),
        jax.ShapeDtypeStruct((NCELLS,), jnp.float32),
    ],
    mesh=plsc.VectorSubcoreMesh(core_axis_name="c", subcore_axis_name="s",
                                num_cores=NC, num_subcores=NS),
    scratch_types=[
        pltpu.VMEM((SC_CH,), jnp.int32),
        pltpu.VMEM((SC_CH,), jnp.int32),
        pltpu.VMEM((SC_CH,), jnp.int32),
        pltpu.VMEM((SC_CH,), jnp.int32),
        pltpu.VMEM((SC_CH,), jnp.float32),
        pltpu.VMEM_SHARED((CCH + SC_CH,), jnp.float32),
        pltpu.SemaphoreType.DMA,
        pltpu.SemaphoreType.DMA,
        pltpu.SemaphoreType.DMA,
        pltpu.SemaphoreType.DMA,
        pltpu.SemaphoreType.DMA,
        pltpu.SemaphoreType.DMA,
        pltpu.SemaphoreType.DMA,
        pltpu.SemaphoreType.DMA,
    ],
)()


def _proj_body(efeat, vfeat, W_in, b_in, W5, b5, W6, b6, W4, b4, W1, b1,
               ke_out, ve_out, qv_out, qe_out):
    fe = jnp.dot(efeat[...], W_in[...], preferred_element_type=jnp.float32) + b_in[...]
    ke = jnp.dot(fe, W5[...], preferred_element_type=jnp.float32) + b5[...]
    ve = jnp.dot(fe, W6[...], preferred_element_type=jnp.float32) + b6[...]
    qv = jnp.dot(vfeat[...], W4[...], preferred_element_type=jnp.float32) + b4[...]
    qe = jnp.dot(fe, W1[...], preferred_element_type=jnp.float32) + b1[...]
    ke_out[...] = ke.astype(jnp.bfloat16)
    ve_out[...] = ve.astype(jnp.bfloat16)
    qv_out[...] = (qv * QSCALE).astype(jnp.bfloat16)
    qe_out[...] = (qe * QSCALE).astype(jnp.bfloat16)


def _stage1_body(qv, ke, ve, c1, ones8, W2, b2, W3, b3,
                 fv_out, kv_out, vv_out):
    s = lax.dot_general(qv[...], ke[...], (((1,), (1,)), ((), ())),
                        preferred_element_type=jnp.float32)
    s = jnp.maximum(s, 0.01 * s)
    a = (jnp.exp2(s) * c1[...]).astype(jnp.bfloat16)
    num = lax.dot_general(a, ve[...], (((1,), (0,)), ((), ())),
                          preferred_element_type=jnp.float32)
    dn = lax.dot_general(a, ones8[...], (((1,), (0,)), ((), ())),
                         preferred_element_type=jnp.float32)
    den = dn[:, 0:1]
    fv = jnp.where(den > 0, num / den, 0.0)
    fv_out[...] = fv
    kv = jnp.dot(fv, W2[...], preferred_element_type=jnp.float32) + b2[...]
    vv = jnp.dot(fv, W3[...], preferred_element_type=jnp.float32) + b3[...]
    kv_out[...] = kv.astype(jnp.bfloat16)
    vv_out[...] = vv.astype(jnp.bfloat16)


def _stage2_body(qe, kv, vv, c2, ones8, fe_out):
    s = lax.dot_general(qe[...], kv[...], (((1,), (1,)), ((), ())),
                        preferred_element_type=jnp.float32)
    s = jnp.maximum(s, 0.01 * s)
    a = (jnp.exp2(s) * c2[...]).astype(jnp.bfloat16)
    num = lax.dot_general(a, vv[...], (((1,), (0,)), ((), ())),
                          preferred_element_type=jnp.float32)
    dn = lax.dot_general(a, ones8[...], (((1,), (0,)), ((), ())),
                         preferred_element_type=jnp.float32)
    den = dn[:, 0:1]
    fe_out[...] = jnp.where(den > 0, num / den, 0.0)


_stage1 = pl.pallas_call(
    _stage1_body,
    grid=(N_V_PAD // BV,),
    in_specs=[
        pl.BlockSpec((BV, D), lambda i: (i, 0)),
        pl.BlockSpec((N_HE_PAD, D), lambda i: (0, 0)),
        pl.BlockSpec((N_HE_PAD, D), lambda i: (0, 0)),
        pl.BlockSpec((BV, N_HE_PAD), lambda i: (i, 0)),
        pl.BlockSpec((N_HE_PAD, 8), lambda i: (0, 0)),
        pl.BlockSpec((D, D), lambda i: (0, 0)),
        pl.BlockSpec((1, D), lambda i: (0, 0)),
        pl.BlockSpec((D, D), lambda i: (0, 0)),
        pl.BlockSpec((1, D), lambda i: (0, 0)),
    ],
    out_specs=[
        pl.BlockSpec((BV, D), lambda i: (i, 0)),
        pl.BlockSpec((BV, D), lambda i: (i, 0)),
        pl.BlockSpec((BV, D), lambda i: (i, 0)),
    ],
    out_shape=[
        jax.ShapeDtypeStruct((N_V_PAD, D), jnp.float32),
        jax.ShapeDtypeStruct((N_V_PAD, D), jnp.bfloat16),
        jax.ShapeDtypeStruct((N_V_PAD, D), jnp.bfloat16),
    ],
)

_stage2 = pl.pallas_call(
    _stage2_body,
    grid=(N_HE_PAD // BE,),
    in_specs=[
        pl.BlockSpec((BE, D), lambda i: (i, 0)),
        pl.BlockSpec((N_V_PAD, D), lambda i: (0, 0)),
        pl.BlockSpec((N_V_PAD, D), lambda i: (0, 0)),
        pl.BlockSpec((BE, N_V_PAD), lambda i: (i, 0)),
        pl.BlockSpec((N_V_PAD, 8), lambda i: (0, 0)),
    ],
    out_specs=pl.BlockSpec((BE, D), lambda i: (i, 0)),
    out_shape=jax.ShapeDtypeStruct((N_HE_PAD, D), jnp.float32),
)


def kernel(vfeat, efeat, inc_node, inc_hedge, W_in, b_in, W1, b1, W2, b2,
           W3, b3, W4, b4, W5, b5, W6, b6):
    f32 = jnp.float32
    bf16 = jnp.bfloat16
    b_in2, b1_2, b2_2, b3_2, b4_2, b5_2, b6_2 = (
        b.reshape(1, D) for b in (b_in, b1, b2, b3, b4, b5, b6))

    rows = N_INC_P // D
    inc_node_p = jnp.pad(inc_node, (0, N_INC_P - N_INC),
                         constant_values=N_V_PAD)
    inc_hedge_p = jnp.pad(inc_hedge, (0, N_INC_P - N_INC),
                          constant_values=N_HE_PAD)
    s1, s2 = _make_sidx()(inc_node_p.reshape(rows, D),
                          inc_hedge_p.reshape(rows, D))
    s1 = s1.reshape(N_CHUNKS, N_INC_P)
    s2 = s2.reshape(N_CHUNKS, N_INC_P)

    zc = jnp.zeros((CCH,), f32)
    ones128 = jnp.ones((SC_CH,), f32)
    c1f, c2f = _count_build(s1, s2, zc, ones128)
    c1 = c1f.reshape(N_V_PAD, N_HE_PAD)
    c2 = c2f.reshape(N_HE_PAD, N_V_PAD)

    ef_p = jnp.pad(efeat, ((0, N_HE_PAD - N_HE), (0, 0)))
    vf_p = jnp.pad(vfeat, ((0, N_V_PAD - N_V), (0, 0)))

    kebf, vebf, qvbf, qebf = pl.pallas_call(
        _proj_body,
        out_shape=[
            jax.ShapeDtypeStruct((N_HE_PAD, D), bf16),
            jax.ShapeDtypeStruct((N_HE_PAD, D), bf16),
            jax.ShapeDtypeStruct((N_V_PAD, D), bf16),
            jax.ShapeDtypeStruct((N_HE_PAD, D), bf16),
        ],
    )(ef_p, vf_p, W_in, b_in2, W5, b5_2, W6, b6_2, W4, b4_2, W1, b1_2)

    ones8e = jnp.ones((N_HE_PAD, 8), bf16)
    ones8v = jnp.ones((N_V_PAD, 8), bf16)

    feat_v, kvbf, vvbf = _stage1(qvbf, kebf, vebf, c1, ones8e,
                                 W2, b2_2, W3, b3_2)
    feat_e2 = _stage2(qebf, kvbf, vvbf, c2, ones8v)

    return feat_v[:N_V], feat_e2[:N_HE]


# speedup vs baseline: 1.7091x; 1.7091x over previous
import functools

import jax
import jax.numpy as jnp
from jax import lax
from jax.experimental import pallas as pl
from jax.experimental.pallas import tpu as pltpu
from jax.experimental.pallas import tpu_sc as plsc

N_V = 10000
N_HE = 2000
N_INC = 320000
D = 128
INV_SQRT_D = 0.08838834764831845
LOG2E = 1.4426950408889634
QSCALE = INV_SQRT_D * LOG2E

N_V_PAD = 10240
N_HE_PAD = 2048
NCELLS = N_V_PAD * N_HE_PAD

NC = 2
NS = 16
N_CHUNKS = 16
CCH = NCELLS // N_CHUNKS
SHARE = CCH // NS
ZB = 8192
SC_CH = 128
N_INC_P = 327680
PER_SUB = N_INC_P // NS
SC_FULL = PER_SUB // SC_CH
NQ = SC_FULL // 4

BV = 512
BE = 128


def _sidx_body(node, hedge, s1_out):
    lane = lax.broadcasted_iota(jnp.int32, node.shape, 1)
    f1 = node[...] * N_HE_PAD + hedge[...]
    for k in range(N_CHUNKS):
        l1 = f1 - k * CCH
        s1_out[k] = jnp.where((l1 >= 0) & (l1 < CCH), l1, CCH + lane)


def _make_sidx():
    rows = N_INC_P // D
    br = rows // 5
    return pl.pallas_call(
        _sidx_body,
        grid=(5,),
        in_specs=[
            pl.BlockSpec((br, D), lambda i: (i, 0)),
            pl.BlockSpec((br, D), lambda i: (i, 0)),
        ],
        out_specs=pl.BlockSpec((N_CHUNKS, br, D), lambda i: (0, i, 0)),
        out_shape=jax.ShapeDtypeStruct((N_CHUNKS, rows, D), jnp.int32),
    )


def _count_body(s1_hbm, zc_hbm, ones_hbm, c1_hbm,
                ib0, ib1, ib2, ib3, ones_v, zbuf, chunk,
                is0, is1, is2, is3, ss0, ss1, ss2, ss3):
    cid = lax.axis_index("c")
    sid = lax.axis_index("s")
    woff = sid * PER_SUB
    ibs = (ib0, ib1, ib2, ib3)
    isems = (is0, is1, is2, is3)
    ssems = (ss0, ss1, ss2, ss3)

    pltpu.sync_copy(ones_hbm, ones_v)
    pltpu.sync_copy(zc_hbm, zbuf)

    for m in range(N_CHUNKS // NC):
        k = cid * (N_CHUNKS // NC) + m
        base = k * CCH

        for z in range(SHARE // ZB):
            pltpu.sync_copy(zbuf,
                            chunk.at[pl.ds(sid * SHARE + z * ZB, ZB)])
        plsc.subcore_barrier()

        def issue_idx(c, buf, sem):
            pltpu.async_copy(s1_hbm.at[k, pl.ds(woff + c * SC_CH, SC_CH)],
                             buf, sem)

        def wait_idx(c, buf, sem):
            pltpu.make_async_copy(
                s1_hbm.at[k, pl.ds(woff + c * SC_CH, SC_CH)],
                buf, sem).wait()

        for j in range(4):
            issue_idx(j, ibs[j], isems[j])

        def quad(q, _):
            c0 = 4 * q
            descs = []
            for j in range(4):
                wait_idx(c0 + j, ibs[j], isems[j])
                descs.append(pltpu.async_copy(ones_v, chunk.at[ibs[j]],
                                              ssems[j], add=True))
            for j in range(4):
                descs[j].wait()

                @pl.when(c0 + j + 4 < SC_FULL)
                def _():
                    issue_idx(c0 + j + 4, ibs[j], isems[j])
            return 0

        lax.fori_loop(0, NQ, quad, 0)

        plsc.subcore_barrier()
        pltpu.sync_copy(chunk.at[pl.ds(sid * SHARE, SHARE)],
                        c1_hbm.at[pl.ds(base + sid * SHARE, SHARE)])


_count_build = functools.partial(
    pl.kernel, _count_body,
    out_type=jax.ShapeDtypeStruct((NCELLS,), jnp.float32),
    mesh=plsc.VectorSubcoreMesh(core_axis_name="c", subcore_axis_name="s",
                                num_cores=NC, num_subcores=NS),
    scratch_types=[
        pltpu.VMEM((SC_CH,), jnp.int32),
        pltpu.VMEM((SC_CH,), jnp.int32),
        pltpu.VMEM((SC_CH,), jnp.int32),
        pltpu.VMEM((SC_CH,), jnp.int32),
        pltpu.VMEM((SC_CH,), jnp.float32),
        pltpu.VMEM((ZB,), jnp.float32),
        pltpu.VMEM_SHARED((CCH + SC_CH,), jnp.float32),
        pltpu.SemaphoreType.DMA,
        pltpu.SemaphoreType.DMA,
        pltpu.SemaphoreType.DMA,
        pltpu.SemaphoreType.DMA,
        pltpu.SemaphoreType.DMA,
        pltpu.SemaphoreType.DMA,
        pltpu.SemaphoreType.DMA,
        pltpu.SemaphoreType.DMA,
    ],
)()


def _proj_body(efeat, vfeat, W_in, b_in, W5, b5, W6, b6, W4, b4, W1, b1,
               ke_out, ve_out, qv_out, qe_out):
    fe = jnp.dot(efeat[...], W_in[...], preferred_element_type=jnp.float32) + b_in[...]
    ke = jnp.dot(fe, W5[...], preferred_element_type=jnp.float32) + b5[...]
    ve = jnp.dot(fe, W6[...], preferred_element_type=jnp.float32) + b6[...]
    qv = jnp.dot(vfeat[...], W4[...], preferred_element_type=jnp.float32) + b4[...]
    qe = jnp.dot(fe, W1[...], preferred_element_type=jnp.float32) + b1[...]
    ke_out[...] = ke.astype(jnp.bfloat16)
    ve_out[...] = ve.astype(jnp.bfloat16)
    qv_out[...] = (qv * QSCALE).astype(jnp.bfloat16)
    qe_out[...] = (qe * QSCALE).astype(jnp.bfloat16)


def _stage1_body(qv, ke, ve, c1, ones8, W2, b2, W3, b3,
                 fv_out, kv_out, vv_out):
    s = lax.dot_general(qv[...], ke[...], (((1,), (1,)), ((), ())),
                        preferred_element_type=jnp.float32)
    s = jnp.maximum(s, 0.01 * s)
    a = (jnp.exp2(s) * c1[...]).astype(jnp.bfloat16)
    num = lax.dot_general(a, ve[...], (((1,), (0,)), ((), ())),
                          preferred_element_type=jnp.float32)
    dn = lax.dot_general(a, ones8[...], (((1,), (0,)), ((), ())),
                         preferred_element_type=jnp.float32)
    den = dn[:, 0:1]
    fv = jnp.where(den > 0, num / den, 0.0)
    fv_out[...] = fv
    kv = jnp.dot(fv, W2[...], preferred_element_type=jnp.float32) + b2[...]
    vv = jnp.dot(fv, W3[...], preferred_element_type=jnp.float32) + b3[...]
    kv_out[...] = kv.astype(jnp.bfloat16)
    vv_out[...] = vv.astype(jnp.bfloat16)


def _stage2_body(qe, kv, vv, c1, ones8, fe_out):
    s = lax.dot_general(kv[...], qe[...], (((1,), (1,)), ((), ())),
                        preferred_element_type=jnp.float32)
    s = jnp.maximum(s, 0.01 * s)
    a = (jnp.exp2(s) * c1[...]).astype(jnp.bfloat16)
    num = lax.dot_general(a, vv[...], (((0,), (0,)), ((), ())),
                          preferred_element_type=jnp.float32)
    dn = lax.dot_general(a, ones8[...], (((0,), (0,)), ((), ())),
                         preferred_element_type=jnp.float32)
    den = dn[:, 0:1]
    fe_out[...] = jnp.where(den > 0, num / den, 0.0)


_stage1 = pl.pallas_call(
    _stage1_body,
    grid=(N_V_PAD // BV,),
    in_specs=[
        pl.BlockSpec((BV, D), lambda i: (i, 0)),
        pl.BlockSpec((N_HE_PAD, D), lambda i: (0, 0)),
        pl.BlockSpec((N_HE_PAD, D), lambda i: (0, 0)),
        pl.BlockSpec((BV, N_HE_PAD), lambda i: (i, 0)),
        pl.BlockSpec((N_HE_PAD, 8), lambda i: (0, 0)),
        pl.BlockSpec((D, D), lambda i: (0, 0)),
        pl.BlockSpec((1, D), lambda i: (0, 0)),
        pl.BlockSpec((D, D), lambda i: (0, 0)),
        pl.BlockSpec((1, D), lambda i: (0, 0)),
    ],
    out_specs=[
        pl.BlockSpec((BV, D), lambda i: (i, 0)),
        pl.BlockSpec((BV, D), lambda i: (i, 0)),
        pl.BlockSpec((BV, D), lambda i: (i, 0)),
    ],
    out_shape=[
        jax.ShapeDtypeStruct((N_V_PAD, D), jnp.float32),
        jax.ShapeDtypeStruct((N_V_PAD, D), jnp.bfloat16),
        jax.ShapeDtypeStruct((N_V_PAD, D), jnp.bfloat16),
    ],
)

_stage2 = pl.pallas_call(
    _stage2_body,
    grid=(N_HE_PAD // BE,),
    in_specs=[
        pl.BlockSpec((BE, D), lambda i: (i, 0)),
        pl.BlockSpec((N_V_PAD, D), lambda i: (0, 0)),
        pl.BlockSpec((N_V_PAD, D), lambda i: (0, 0)),
        pl.BlockSpec((N_V_PAD, BE), lambda i: (0, i)),
        pl.BlockSpec((N_V_PAD, 8), lambda i: (0, 0)),
    ],
    out_specs=pl.BlockSpec((BE, D), lambda i: (i, 0)),
    out_shape=jax.ShapeDtypeStruct((N_HE_PAD, D), jnp.float32),
)


def kernel(vfeat, efeat, inc_node, inc_hedge, W_in, b_in, W1, b1, W2, b2,
           W3, b3, W4, b4, W5, b5, W6, b6):
    f32 = jnp.float32
    bf16 = jnp.bfloat16
    b_in2, b1_2, b2_2, b3_2, b4_2, b5_2, b6_2 = (
        b.reshape(1, D) for b in (b_in, b1, b2, b3, b4, b5, b6))

    rows = N_INC_P // D
    inc_node_p = jnp.pad(inc_node, (0, N_INC_P - N_INC),
                         constant_values=N_V_PAD)
    inc_hedge_p = jnp.pad(inc_hedge, (0, N_INC_P - N_INC),
                          constant_values=N_HE_PAD)
    s1 = _make_sidx()(inc_node_p.reshape(rows, D),
                      inc_hedge_p.reshape(rows, D))
    s1 = s1.reshape(N_CHUNKS, N_INC_P)

    zc = jnp.zeros((ZB,), f32)
    ones128 = jnp.ones((SC_CH,), f32)
    c1f = _count_build(s1, zc, ones128)
    c1 = c1f.reshape(N_V_PAD, N_HE_PAD)

    ef_p = jnp.pad(efeat, ((0, N_HE_PAD - N_HE), (0, 0)))
    vf_p = jnp.pad(vfeat, ((0, N_V_PAD - N_V), (0, 0)))

    kebf, vebf, qvbf, qebf = pl.pallas_call(
        _proj_body,
        out_shape=[
            jax.ShapeDtypeStruct((N_HE_PAD, D), bf16),
            jax.ShapeDtypeStruct((N_HE_PAD, D), bf16),
            jax.ShapeDtypeStruct((N_V_PAD, D), bf16),
            jax.ShapeDtypeStruct((N_HE_PAD, D), bf16),
        ],
    )(ef_p, vf_p, W_in, b_in2, W5, b5_2, W6, b6_2, W4, b4_2, W1, b1_2)

    ones8e = jnp.ones((N_HE_PAD, 8), bf16)
    ones8v = jnp.ones((N_V_PAD, 8), bf16)

    feat_v, kvbf, vvbf = _stage1(qvbf, kebf, vebf, c1, ones8e,
                                 W2, b2_2, W3, b3_2)
    feat_e2 = _stage2(qebf, kvbf, vvbf, c1, ones8v)

    return feat_v[:N_V], feat_e2[:N_HE]


# speedup vs baseline: 2.0188x; 1.1812x over previous
import functools

import jax
import jax.numpy as jnp
from jax import lax
from jax.experimental import pallas as pl
from jax.experimental.pallas import tpu as pltpu
from jax.experimental.pallas import tpu_sc as plsc

N_V = 10000
N_HE = 2000
N_INC = 320000
D = 128
INV_SQRT_D = 0.08838834764831845
LOG2E = 1.4426950408889634
QSCALE = INV_SQRT_D * LOG2E

N_V_PAD = 10240
N_HE_PAD = 2048
NCELLS = N_V_PAD * N_HE_PAD

NC = 2
NS = 16
N_CHUNKS = 16
CCH = NCELLS // N_CHUNKS
SHARE = CCH // NS
ZB = 8192
SC_CH = 256
N_INC_P = 327680
PER_SUB = N_INC_P // NS
SC_FULL = PER_SUB // SC_CH
NQ = SC_FULL // 4

BV = 512
BE = 128


def _sidx_body(node, hedge, s1_out):
    lane = lax.broadcasted_iota(jnp.int32, node.shape, 1)
    f1 = node[...] * N_HE_PAD + hedge[...]
    for k in range(N_CHUNKS):
        l1 = f1 - k * CCH
        s1_out[k] = jnp.where((l1 >= 0) & (l1 < CCH), l1, CCH + lane)


def _make_sidx():
    rows = N_INC_P // D
    br = rows // 5
    return pl.pallas_call(
        _sidx_body,
        grid=(5,),
        in_specs=[
            pl.BlockSpec((br, D), lambda i: (i, 0)),
            pl.BlockSpec((br, D), lambda i: (i, 0)),
        ],
        out_specs=pl.BlockSpec((N_CHUNKS, br, D), lambda i: (0, i, 0)),
        out_shape=jax.ShapeDtypeStruct((N_CHUNKS, rows, D), jnp.int32),
    )


def _count_body(s1_hbm, zc_hbm, ones_hbm, c1_hbm,
                ib0, ib1, ib2, ib3, ones_v, zbuf, chunk,
                is0, is1, is2, is3, ss0, ss1, ss2, ss3):
    cid = lax.axis_index("c")
    sid = lax.axis_index("s")
    woff = sid * PER_SUB
    ibs = (ib0, ib1, ib2, ib3)
    isems = (is0, is1, is2, is3)
    ssems = (ss0, ss1, ss2, ss3)

    pltpu.sync_copy(ones_hbm, ones_v)
    pltpu.sync_copy(zc_hbm, zbuf)

    for m in range(N_CHUNKS // NC):
        k = cid * (N_CHUNKS // NC) + m
        base = k * CCH

        for z in range(SHARE // ZB):
            pltpu.sync_copy(zbuf,
                            chunk.at[pl.ds(sid * SHARE + z * ZB, ZB)])
        plsc.subcore_barrier()

        def issue_idx(c, buf, sem):
            pltpu.async_copy(s1_hbm.at[k, pl.ds(woff + c * SC_CH, SC_CH)],
                             buf, sem)

        def wait_idx(c, buf, sem):
            pltpu.make_async_copy(
                s1_hbm.at[k, pl.ds(woff + c * SC_CH, SC_CH)],
                buf, sem).wait()

        for j in range(4):
            issue_idx(j, ibs[j], isems[j])

        def quad(q, _):
            c0 = 4 * q
            descs = []
            for j in range(4):
                wait_idx(c0 + j, ibs[j], isems[j])
                descs.append(pltpu.async_copy(ones_v, chunk.at[ibs[j]],
                                              ssems[j], add=True))
            for j in range(4):
                descs[j].wait()

                @pl.when(c0 + j + 4 < SC_FULL)
                def _():
                    issue_idx(c0 + j + 4, ibs[j], isems[j])
            return 0

        lax.fori_loop(0, NQ, quad, 0)

        plsc.subcore_barrier()
        pltpu.sync_copy(chunk.at[pl.ds(sid * SHARE, SHARE)],
                        c1_hbm.at[pl.ds(base + sid * SHARE, SHARE)])


_count_build = functools.partial(
    pl.kernel, _count_body,
    out_type=jax.ShapeDtypeStruct((NCELLS,), jnp.float32),
    mesh=plsc.VectorSubcoreMesh(core_axis_name="c", subcore_axis_name="s",
                                num_cores=NC, num_subcores=NS),
    scratch_types=[
        pltpu.VMEM((SC_CH,), jnp.int32),
        pltpu.VMEM((SC_CH,), jnp.int32),
        pltpu.VMEM((SC_CH,), jnp.int32),
        pltpu.VMEM((SC_CH,), jnp.int32),
        pltpu.VMEM((SC_CH,), jnp.float32),
        pltpu.VMEM((ZB,), jnp.float32),
        pltpu.VMEM_SHARED((CCH + SC_CH,), jnp.float32),
        pltpu.SemaphoreType.DMA,
        pltpu.SemaphoreType.DMA,
        pltpu.SemaphoreType.DMA,
        pltpu.SemaphoreType.DMA,
        pltpu.SemaphoreType.DMA,
        pltpu.SemaphoreType.DMA,
        pltpu.SemaphoreType.DMA,
        pltpu.SemaphoreType.DMA,
    ],
)()


def _proj_body(efeat, vfeat, W_in, b_in, W5, b5, W6, b6, W4, b4, W1, b1,
               ke_out, ve_out, qv_out, qe_out):
    fe = jnp.dot(efeat[...], W_in[...], preferred_element_type=jnp.float32) + b_in[...]
    ke = jnp.dot(fe, W5[...], preferred_element_type=jnp.float32) + b5[...]
    ve = jnp.dot(fe, W6[...], preferred_element_type=jnp.float32) + b6[...]
    qv = jnp.dot(vfeat[...], W4[...], preferred_element_type=jnp.float32) + b4[...]
    qe = jnp.dot(fe, W1[...], preferred_element_type=jnp.float32) + b1[...]
    ke_out[...] = ke.astype(jnp.bfloat16)
    ve_out[...] = ve.astype(jnp.bfloat16)
    qv_out[...] = (qv * QSCALE).astype(jnp.bfloat16)
    qe_out[...] = (qe * QSCALE).astype(jnp.bfloat16)


def _stage1_body(qv, ke, ve, c1, ones8, W2, b2, W3, b3,
                 fv_out, kv_out, vv_out):
    s = lax.dot_general(qv[...], ke[...], (((1,), (1,)), ((), ())),
                        preferred_element_type=jnp.float32)
    s = jnp.maximum(s, 0.01 * s)
    a = (jnp.exp2(s) * c1[...]).astype(jnp.bfloat16)
    num = lax.dot_general(a, ve[...], (((1,), (0,)), ((), ())),
                          preferred_element_type=jnp.float32)
    dn = lax.dot_general(a, ones8[...], (((1,), (0,)), ((), ())),
                         preferred_element_type=jnp.float32)
    den = dn[:, 0:1]
    fv = jnp.where(den > 0, num / den, 0.0)
    fv_out[...] = fv
    kv = jnp.dot(fv, W2[...], preferred_element_type=jnp.float32) + b2[...]
    vv = jnp.dot(fv, W3[...], preferred_element_type=jnp.float32) + b3[...]
    kv_out[...] = kv.astype(jnp.bfloat16)
    vv_out[...] = vv.astype(jnp.bfloat16)


def _stage2_body(qe, kv, vv, c1, ones8, fe_out):
    s = lax.dot_general(kv[...], qe[...], (((1,), (1,)), ((), ())),
                        preferred_element_type=jnp.float32)
    s = jnp.maximum(s, 0.01 * s)
    a = (jnp.exp2(s) * c1[...]).astype(jnp.bfloat16)
    num = lax.dot_general(a, vv[...], (((0,), (0,)), ((), ())),
                          preferred_element_type=jnp.float32)
    dn = lax.dot_general(a, ones8[...], (((0,), (0,)), ((), ())),
                         preferred_element_type=jnp.float32)
    den = dn[:, 0:1]
    fe_out[...] = jnp.where(den > 0, num / den, 0.0)


_stage1 = pl.pallas_call(
    _stage1_body,
    grid=(N_V_PAD // BV,),
    in_specs=[
        pl.BlockSpec((BV, D), lambda i: (i, 0)),
        pl.BlockSpec((N_HE_PAD, D), lambda i: (0, 0)),
        pl.BlockSpec((N_HE_PAD, D), lambda i: (0, 0)),
        pl.BlockSpec((BV, N_HE_PAD), lambda i: (i, 0)),
        pl.BlockSpec((N_HE_PAD, 8), lambda i: (0, 0)),
        pl.BlockSpec((D, D), lambda i: (0, 0)),
        pl.BlockSpec((1, D), lambda i: (0, 0)),
        pl.BlockSpec((D, D), lambda i: (0, 0)),
        pl.BlockSpec((1, D), lambda i: (0, 0)),
    ],
    out_specs=[
        pl.BlockSpec((BV, D), lambda i: (i, 0)),
        pl.BlockSpec((BV, D), lambda i: (i, 0)),
        pl.BlockSpec((BV, D), lambda i: (i, 0)),
    ],
    out_shape=[
        jax.ShapeDtypeStruct((N_V_PAD, D), jnp.float32),
        jax.ShapeDtypeStruct((N_V_PAD, D), jnp.bfloat16),
        jax.ShapeDtypeStruct((N_V_PAD, D), jnp.bfloat16),
    ],
)

_stage2 = pl.pallas_call(
    _stage2_body,
    grid=(N_HE_PAD // BE,),
    in_specs=[
        pl.BlockSpec((BE, D), lambda i: (i, 0)),
        pl.BlockSpec((N_V_PAD, D), lambda i: (0, 0)),
        pl.BlockSpec((N_V_PAD, D), lambda i: (0, 0)),
        pl.BlockSpec((N_V_PAD, BE), lambda i: (0, i)),
        pl.BlockSpec((N_V_PAD, 8), lambda i: (0, 0)),
    ],
    out_specs=pl.BlockSpec((BE, D), lambda i: (i, 0)),
    out_shape=jax.ShapeDtypeStruct((N_HE_PAD, D), jnp.float32),
)


def kernel(vfeat, efeat, inc_node, inc_hedge, W_in, b_in, W1, b1, W2, b2,
           W3, b3, W4, b4, W5, b5, W6, b6):
    f32 = jnp.float32
    bf16 = jnp.bfloat16
    b_in2, b1_2, b2_2, b3_2, b4_2, b5_2, b6_2 = (
        b.reshape(1, D) for b in (b_in, b1, b2, b3, b4, b5, b6))

    rows = N_INC_P // D
    inc_node_p = jnp.pad(inc_node, (0, N_INC_P - N_INC),
                         constant_values=N_V_PAD)
    inc_hedge_p = jnp.pad(inc_hedge, (0, N_INC_P - N_INC),
                          constant_values=N_HE_PAD)
    s1 = _make_sidx()(inc_node_p.reshape(rows, D),
                      inc_hedge_p.reshape(rows, D))
    s1 = s1.reshape(N_CHUNKS, N_INC_P)

    zc = jnp.zeros((ZB,), f32)
    ones128 = jnp.ones((SC_CH,), f32)
    c1f = _count_build(s1, zc, ones128)
    c1 = c1f.reshape(N_V_PAD, N_HE_PAD)

    ef_p = jnp.pad(efeat, ((0, N_HE_PAD - N_HE), (0, 0)))
    vf_p = jnp.pad(vfeat, ((0, N_V_PAD - N_V), (0, 0)))

    kebf, vebf, qvbf, qebf = pl.pallas_call(
        _proj_body,
        out_shape=[
            jax.ShapeDtypeStruct((N_HE_PAD, D), bf16),
            jax.ShapeDtypeStruct((N_HE_PAD, D), bf16),
            jax.ShapeDtypeStruct((N_V_PAD, D), bf16),
            jax.ShapeDtypeStruct((N_HE_PAD, D), bf16),
        ],
    )(ef_p, vf_p, W_in, b_in2, W5, b5_2, W6, b6_2, W4, b4_2, W1, b1_2)

    ones8e = jnp.ones((N_HE_PAD, 8), bf16)
    ones8v = jnp.ones((N_V_PAD, 8), bf16)

    feat_v, kvbf, vvbf = _stage1(qvbf, kebf, vebf, c1, ones8e,
                                 W2, b2_2, W3, b3_2)
    feat_e2 = _stage2(qebf, kvbf, vvbf, c1, ones8v)

    return feat_v[:N_V], feat_e2[:N_HE]


# speedup vs baseline: 2.2208x; 1.1000x over previous
import functools

import jax
import jax.numpy as jnp
from jax import lax
from jax.experimental import pallas as pl
from jax.experimental.pallas import tpu as pltpu
from jax.experimental.pallas import tpu_sc as plsc

N_V = 10000
N_HE = 2000
N_INC = 320000
D = 128
INV_SQRT_D = 0.08838834764831845
LOG2E = 1.4426950408889634
QSCALE = INV_SQRT_D * LOG2E

N_V_PAD = 10240
N_HE_PAD = 2048
NCELLS = N_V_PAD * N_HE_PAD

NC = 2
NS = 16
N_CHUNKS = 16
CCH = NCELLS // N_CHUNKS
SHARE = CCH // NS
ZB = 8192
SC_CH = 512
N_INC_P = 327680
PER_SUB = N_INC_P // NS
SC_FULL = PER_SUB // SC_CH
NQ = SC_FULL // 4

BV = 512
BE = 128


def _sidx_body(node, hedge, s1_out):
    lane = lax.broadcasted_iota(jnp.int32, node.shape, 1)
    f1 = node[...] * N_HE_PAD + hedge[...]
    for k in range(N_CHUNKS):
        l1 = f1 - k * CCH
        s1_out[k] = jnp.where((l1 >= 0) & (l1 < CCH), l1, CCH + lane)


def _make_sidx():
    rows = N_INC_P // D
    br = rows // 5
    return pl.pallas_call(
        _sidx_body,
        grid=(5,),
        in_specs=[
            pl.BlockSpec((br, D), lambda i: (i, 0)),
            pl.BlockSpec((br, D), lambda i: (i, 0)),
        ],
        out_specs=pl.BlockSpec((N_CHUNKS, br, D), lambda i: (0, i, 0)),
        out_shape=jax.ShapeDtypeStruct((N_CHUNKS, rows, D), jnp.int32),
    )


def _count_body(s1_hbm, zc_hbm, ones_hbm, c1_hbm,
                ib0, ib1, ib2, ib3, ones_v, zbuf, chunk,
                is0, is1, is2, is3, ss0, ss1, ss2, ss3):
    cid = lax.axis_index("c")
    sid = lax.axis_index("s")
    woff = sid * PER_SUB
    ibs = (ib0, ib1, ib2, ib3)
    isems = (is0, is1, is2, is3)
    ssems = (ss0, ss1, ss2, ss3)

    pltpu.sync_copy(ones_hbm, ones_v)
    pltpu.sync_copy(zc_hbm, zbuf)

    for m in range(N_CHUNKS // NC):
        k = cid * (N_CHUNKS // NC) + m
        base = k * CCH

        for z in range(SHARE // ZB):
            pltpu.sync_copy(zbuf,
                            chunk.at[pl.ds(sid * SHARE + z * ZB, ZB)])
        plsc.subcore_barrier()

        def issue_idx(c, buf, sem):
            pltpu.async_copy(s1_hbm.at[k, pl.ds(woff + c * SC_CH, SC_CH)],
                             buf, sem)

        def wait_idx(c, buf, sem):
            pltpu.make_async_copy(
                s1_hbm.at[k, pl.ds(woff + c * SC_CH, SC_CH)],
                buf, sem).wait()

        for j in range(4):
            issue_idx(j, ibs[j], isems[j])

        def quad(q, _):
            c0 = 4 * q
            descs = []
            for j in range(4):
                wait_idx(c0 + j, ibs[j], isems[j])
                descs.append(pltpu.async_copy(ones_v, chunk.at[ibs[j]],
                                              ssems[j], add=True))
            for j in range(4):
                descs[j].wait()

                @pl.when(c0 + j + 4 < SC_FULL)
                def _():
                    issue_idx(c0 + j + 4, ibs[j], isems[j])
            return 0

        lax.fori_loop(0, NQ, quad, 0)

        plsc.subcore_barrier()
        pltpu.sync_copy(chunk.at[pl.ds(sid * SHARE, SHARE)],
                        c1_hbm.at[pl.ds(base + sid * SHARE, SHARE)])


_count_build = functools.partial(
    pl.kernel, _count_body,
    out_type=jax.ShapeDtypeStruct((NCELLS,), jnp.float32),
    mesh=plsc.VectorSubcoreMesh(core_axis_name="c", subcore_axis_name="s",
                                num_cores=NC, num_subcores=NS),
    scratch_types=[
        pltpu.VMEM((SC_CH,), jnp.int32),
        pltpu.VMEM((SC_CH,), jnp.int32),
        pltpu.VMEM((SC_CH,), jnp.int32),
        pltpu.VMEM((SC_CH,), jnp.int32),
        pltpu.VMEM((SC_CH,), jnp.float32),
        pltpu.VMEM((ZB,), jnp.float32),
        pltpu.VMEM_SHARED((CCH + SC_CH,), jnp.float32),
        pltpu.SemaphoreType.DMA,
        pltpu.SemaphoreType.DMA,
        pltpu.SemaphoreType.DMA,
        pltpu.SemaphoreType.DMA,
        pltpu.SemaphoreType.DMA,
        pltpu.SemaphoreType.DMA,
        pltpu.SemaphoreType.DMA,
        pltpu.SemaphoreType.DMA,
    ],
)()


def _proj_body(efeat, vfeat, W_in, b_in, W5, b5, W6, b6, W4, b4, W1, b1,
               ke_out, ve_out, qv_out, qe_out):
    fe = jnp.dot(efeat[...], W_in[...], preferred_element_type=jnp.float32) + b_in[...]
    ke = jnp.dot(fe, W5[...], preferred_element_type=jnp.float32) + b5[...]
    ve = jnp.dot(fe, W6[...], preferred_element_type=jnp.float32) + b6[...]
    qv = jnp.dot(vfeat[...], W4[...], preferred_element_type=jnp.float32) + b4[...]
    qe = jnp.dot(fe, W1[...], preferred_element_type=jnp.float32) + b1[...]
    ke_out[...] = ke.astype(jnp.bfloat16)
    ve_out[...] = ve.astype(jnp.bfloat16)
    qv_out[...] = (qv * QSCALE).astype(jnp.bfloat16)
    qe_out[...] = (qe * QSCALE).astype(jnp.bfloat16)


def _stage1_body(qv, ke, ve, c1, ones8, W2, b2, W3, b3,
                 fv_out, kv_out, vv_out):
    s = lax.dot_general(qv[...], ke[...], (((1,), (1,)), ((), ())),
                        preferred_element_type=jnp.float32)
    s = jnp.maximum(s, 0.01 * s)
    a = (jnp.exp2(s) * c1[...]).astype(jnp.bfloat16)
    num = lax.dot_general(a, ve[...], (((1,), (0,)), ((), ())),
                          preferred_element_type=jnp.float32)
    dn = lax.dot_general(a, ones8[...], (((1,), (0,)), ((), ())),
                         preferred_element_type=jnp.float32)
    den = dn[:, 0:1]
    fv = jnp.where(den > 0, num / den, 0.0)
    fv_out[...] = fv
    kv = jnp.dot(fv, W2[...], preferred_element_type=jnp.float32) + b2[...]
    vv = jnp.dot(fv, W3[...], preferred_element_type=jnp.float32) + b3[...]
    kv_out[...] = kv.astype(jnp.bfloat16)
    vv_out[...] = vv.astype(jnp.bfloat16)


def _stage2_body(qe, kv, vv, c1, ones8, fe_out):
    s = lax.dot_general(kv[...], qe[...], (((1,), (1,)), ((), ())),
                        preferred_element_type=jnp.float32)
    s = jnp.maximum(s, 0.01 * s)
    a = (jnp.exp2(s) * c1[...]).astype(jnp.bfloat16)
    num = lax.dot_general(a, vv[...], (((0,), (0,)), ((), ())),
                          preferred_element_type=jnp.float32)
    dn = lax.dot_general(a, ones8[...], (((0,), (0,)), ((), ())),
                         preferred_element_type=jnp.float32)
    den = dn[:, 0:1]
    fe_out[...] = jnp.where(den > 0, num / den, 0.0)


_stage1 = pl.pallas_call(
    _stage1_body,
    grid=(N_V_PAD // BV,),
    in_specs=[
        pl.BlockSpec((BV, D), lambda i: (i, 0)),
        pl.BlockSpec((N_HE_PAD, D), lambda i: (0, 0)),
        pl.BlockSpec((N_HE_PAD, D), lambda i: (0, 0)),
        pl.BlockSpec((BV, N_HE_PAD), lambda i: (i, 0)),
        pl.BlockSpec((N_HE_PAD, 8), lambda i: (0, 0)),
        pl.BlockSpec((D, D), lambda i: (0, 0)),
        pl.BlockSpec((1, D), lambda i: (0, 0)),
        pl.BlockSpec((D, D), lambda i: (0, 0)),
        pl.BlockSpec((1, D), lambda i: (0, 0)),
    ],
    out_specs=[
        pl.BlockSpec((BV, D), lambda i: (i, 0)),
        pl.BlockSpec((BV, D), lambda i: (i, 0)),
        pl.BlockSpec((BV, D), lambda i: (i, 0)),
    ],
    out_shape=[
        jax.ShapeDtypeStruct((N_V_PAD, D), jnp.float32),
        jax.ShapeDtypeStruct((N_V_PAD, D), jnp.bfloat16),
        jax.ShapeDtypeStruct((N_V_PAD, D), jnp.bfloat16),
    ],
)

_stage2 = pl.pallas_call(
    _stage2_body,
    grid=(N_HE_PAD // BE,),
    in_specs=[
        pl.BlockSpec((BE, D), lambda i: (i, 0)),
        pl.BlockSpec((N_V_PAD, D), lambda i: (0, 0)),
        pl.BlockSpec((N_V_PAD, D), lambda i: (0, 0)),
        pl.BlockSpec((N_V_PAD, BE), lambda i: (0, i)),
        pl.BlockSpec((N_V_PAD, 8), lambda i: (0, 0)),
    ],
    out_specs=pl.BlockSpec((BE, D), lambda i: (i, 0)),
    out_shape=jax.ShapeDtypeStruct((N_HE_PAD, D), jnp.float32),
)


def kernel(vfeat, efeat, inc_node, inc_hedge, W_in, b_in, W1, b1, W2, b2,
           W3, b3, W4, b4, W5, b5, W6, b6):
    f32 = jnp.float32
    bf16 = jnp.bfloat16
    b_in2, b1_2, b2_2, b3_2, b4_2, b5_2, b6_2 = (
        b.reshape(1, D) for b in (b_in, b1, b2, b3, b4, b5, b6))

    rows = N_INC_P // D
    inc_node_p = jnp.pad(inc_node, (0, N_INC_P - N_INC),
                         constant_values=N_V_PAD)
    inc_hedge_p = jnp.pad(inc_hedge, (0, N_INC_P - N_INC),
                          constant_values=N_HE_PAD)
    s1 = _make_sidx()(inc_node_p.reshape(rows, D),
                      inc_hedge_p.reshape(rows, D))
    s1 = s1.reshape(N_CHUNKS, N_INC_P)

    zc = jnp.zeros((ZB,), f32)
    ones128 = jnp.ones((SC_CH,), f32)
    c1f = _count_build(s1, zc, ones128)
    c1 = c1f.reshape(N_V_PAD, N_HE_PAD)

    ef_p = jnp.pad(efeat, ((0, N_HE_PAD - N_HE), (0, 0)))
    vf_p = jnp.pad(vfeat, ((0, N_V_PAD - N_V), (0, 0)))

    kebf, vebf, qvbf, qebf = pl.pallas_call(
        _proj_body,
        out_shape=[
            jax.ShapeDtypeStruct((N_HE_PAD, D), bf16),
            jax.ShapeDtypeStruct((N_HE_PAD, D), bf16),
            jax.ShapeDtypeStruct((N_V_PAD, D), bf16),
            jax.ShapeDtypeStruct((N_HE_PAD, D), bf16),
        ],
    )(ef_p, vf_p, W_in, b_in2, W5, b5_2, W6, b6_2, W4, b4_2, W1, b1_2)

    ones8e = jnp.ones((N_HE_PAD, 8), bf16)
    ones8v = jnp.ones((N_V_PAD, 8), bf16)

    feat_v, kvbf, vvbf = _stage1(qvbf, kebf, vebf, c1, ones8e,
                                 W2, b2_2, W3, b3_2)
    feat_e2 = _stage2(qebf, kvbf, vvbf, c1, ones8v)

    return feat_v[:N_V], feat_e2[:N_HE]


# speedup vs baseline: 2.3358x; 1.0518x over previous
import functools

import jax
import jax.numpy as jnp
from jax import lax
from jax.experimental import pallas as pl
from jax.experimental.pallas import tpu as pltpu
from jax.experimental.pallas import tpu_sc as plsc

N_V = 10000
N_HE = 2000
N_INC = 320000
D = 128
INV_SQRT_D = 0.08838834764831845
LOG2E = 1.4426950408889634
QSCALE = INV_SQRT_D * LOG2E

N_V_PAD = 10240
N_HE_PAD = 2048
NCELLS = N_V_PAD * N_HE_PAD

NC = 2
NS = 16
N_CHUNKS = 16
CCH = NCELLS // N_CHUNKS
SHARE = CCH // NS
ZB = 8192
SC_CH = 1024
N_INC_P = 327680
PER_SUB = N_INC_P // NS
SC_FULL = PER_SUB // SC_CH
NQ = SC_FULL // 4

BV = 512
BE = 128


def _sidx_body(node, hedge, s1_out):
    lane = lax.broadcasted_iota(jnp.int32, node.shape, 1)
    f1 = node[...] * N_HE_PAD + hedge[...]
    for k in range(N_CHUNKS):
        l1 = f1 - k * CCH
        s1_out[k] = jnp.where((l1 >= 0) & (l1 < CCH), l1, CCH + lane)


def _make_sidx():
    rows = N_INC_P // D
    br = rows // 5
    return pl.pallas_call(
        _sidx_body,
        grid=(5,),
        in_specs=[
            pl.BlockSpec((br, D), lambda i: (i, 0)),
            pl.BlockSpec((br, D), lambda i: (i, 0)),
        ],
        out_specs=pl.BlockSpec((N_CHUNKS, br, D), lambda i: (0, i, 0)),
        out_shape=jax.ShapeDtypeStruct((N_CHUNKS, rows, D), jnp.int32),
    )


def _count_body(s1_hbm, zc_hbm, ones_hbm, c1_hbm,
                ib0, ib1, ib2, ib3, ones_v, zbuf, chunk,
                is0, is1, is2, is3, ss0, ss1, ss2, ss3):
    cid = lax.axis_index("c")
    sid = lax.axis_index("s")
    woff = sid * PER_SUB
    ibs = (ib0, ib1, ib2, ib3)
    isems = (is0, is1, is2, is3)
    ssems = (ss0, ss1, ss2, ss3)

    pltpu.sync_copy(ones_hbm, ones_v)
    pltpu.sync_copy(zc_hbm, zbuf)

    for m in range(N_CHUNKS // NC):
        k = cid * (N_CHUNKS // NC) + m
        base = k * CCH

        for z in range(SHARE // ZB):
            pltpu.sync_copy(zbuf,
                            chunk.at[pl.ds(sid * SHARE + z * ZB, ZB)])
        plsc.subcore_barrier()

        def issue_idx(c, buf, sem):
            pltpu.async_copy(s1_hbm.at[k, pl.ds(woff + c * SC_CH, SC_CH)],
                             buf, sem)

        def wait_idx(c, buf, sem):
            pltpu.make_async_copy(
                s1_hbm.at[k, pl.ds(woff + c * SC_CH, SC_CH)],
                buf, sem).wait()

        for j in range(4):
            issue_idx(j, ibs[j], isems[j])

        def quad(q, _):
            c0 = 4 * q
            descs = []
            for j in range(4):
                wait_idx(c0 + j, ibs[j], isems[j])
                descs.append(pltpu.async_copy(ones_v, chunk.at[ibs[j]],
                                              ssems[j], add=True))
            for j in range(4):
                descs[j].wait()

                @pl.when(c0 + j + 4 < SC_FULL)
                def _():
                    issue_idx(c0 + j + 4, ibs[j], isems[j])
            return 0

        lax.fori_loop(0, NQ, quad, 0)

        plsc.subcore_barrier()
        pltpu.sync_copy(chunk.at[pl.ds(sid * SHARE, SHARE)],
                        c1_hbm.at[pl.ds(base + sid * SHARE, SHARE)])


_count_build = functools.partial(
    pl.kernel, _count_body,
    out_type=jax.ShapeDtypeStruct((NCELLS,), jnp.float32),
    mesh=plsc.VectorSubcoreMesh(core_axis_name="c", subcore_axis_name="s",
                                num_cores=NC, num_subcores=NS),
    scratch_types=[
        pltpu.VMEM((SC_CH,), jnp.int32),
        pltpu.VMEM((SC_CH,), jnp.int32),
        pltpu.VMEM((SC_CH,), jnp.int32),
        pltpu.VMEM((SC_CH,), jnp.int32),
        pltpu.VMEM((SC_CH,), jnp.float32),
        pltpu.VMEM((ZB,), jnp.float32),
        pltpu.VMEM_SHARED((CCH + SC_CH,), jnp.float32),
        pltpu.SemaphoreType.DMA,
        pltpu.SemaphoreType.DMA,
        pltpu.SemaphoreType.DMA,
        pltpu.SemaphoreType.DMA,
        pltpu.SemaphoreType.DMA,
        pltpu.SemaphoreType.DMA,
        pltpu.SemaphoreType.DMA,
        pltpu.SemaphoreType.DMA,
    ],
)()


def _proj_body(efeat, vfeat, W_in, b_in, W5, b5, W6, b6, W4, b4, W1, b1,
               ke_out, ve_out, qv_out, qe_out):
    fe = jnp.dot(efeat[...], W_in[...], preferred_element_type=jnp.float32) + b_in[...]
    ke = jnp.dot(fe, W5[...], preferred_element_type=jnp.float32) + b5[...]
    ve = jnp.dot(fe, W6[...], preferred_element_type=jnp.float32) + b6[...]
    qv = jnp.dot(vfeat[...], W4[...], preferred_element_type=jnp.float32) + b4[...]
    qe = jnp.dot(fe, W1[...], preferred_element_type=jnp.float32) + b1[...]
    ke_out[...] = ke.astype(jnp.bfloat16)
    ve_out[...] = ve.astype(jnp.bfloat16)
    qv_out[...] = (qv * QSCALE).astype(jnp.bfloat16)
    qe_out[...] = (qe * QSCALE).astype(jnp.bfloat16)


def _stage1_body(qv, ke, ve, c1, ones8, W2, b2, W3, b3,
                 fv_out, kv_out, vv_out):
    s = lax.dot_general(qv[...], ke[...], (((1,), (1,)), ((), ())),
                        preferred_element_type=jnp.float32)
    s = jnp.maximum(s, 0.01 * s)
    a = (jnp.exp2(s) * c1[...]).astype(jnp.bfloat16)
    num = lax.dot_general(a, ve[...], (((1,), (0,)), ((), ())),
                          preferred_element_type=jnp.float32)
    dn = lax.dot_general(a, ones8[...], (((1,), (0,)), ((), ())),
                         preferred_element_type=jnp.float32)
    den = dn[:, 0:1]
    fv = jnp.where(den > 0, num / den, 0.0)
    fv_out[...] = fv
    kv = jnp.dot(fv, W2[...], preferred_element_type=jnp.float32) + b2[...]
    vv = jnp.dot(fv, W3[...], preferred_element_type=jnp.float32) + b3[...]
    kv_out[...] = kv.astype(jnp.bfloat16)
    vv_out[...] = vv.astype(jnp.bfloat16)


def _stage2_body(qe, kv, vv, c1, ones8, fe_out):
    s = lax.dot_general(kv[...], qe[...], (((1,), (1,)), ((), ())),
                        preferred_element_type=jnp.float32)
    s = jnp.maximum(s, 0.01 * s)
    a = (jnp.exp2(s) * c1[...]).astype(jnp.bfloat16)
    num = lax.dot_general(a, vv[...], (((0,), (0,)), ((), ())),
                          preferred_element_type=jnp.float32)
    dn = lax.dot_general(a, ones8[...], (((0,), (0,)), ((), ())),
                         preferred_element_type=jnp.float32)
    den = dn[:, 0:1]
    fe_out[...] = jnp.where(den > 0, num / den, 0.0)


_stage1 = pl.pallas_call(
    _stage1_body,
    grid=(N_V_PAD // BV,),
    in_specs=[
        pl.BlockSpec((BV, D), lambda i: (i, 0)),
        pl.BlockSpec((N_HE_PAD, D), lambda i: (0, 0)),
        pl.BlockSpec((N_HE_PAD, D), lambda i: (0, 0)),
        pl.BlockSpec((BV, N_HE_PAD), lambda i: (i, 0)),
        pl.BlockSpec((N_HE_PAD, 8), lambda i: (0, 0)),
        pl.BlockSpec((D, D), lambda i: (0, 0)),
        pl.BlockSpec((1, D), lambda i: (0, 0)),
        pl.BlockSpec((D, D), lambda i: (0, 0)),
        pl.BlockSpec((1, D), lambda i: (0, 0)),
    ],
    out_specs=[
        pl.BlockSpec((BV, D), lambda i: (i, 0)),
        pl.BlockSpec((BV, D), lambda i: (i, 0)),
        pl.BlockSpec((BV, D), lambda i: (i, 0)),
    ],
    out_shape=[
        jax.ShapeDtypeStruct((N_V_PAD, D), jnp.float32),
        jax.ShapeDtypeStruct((N_V_PAD, D), jnp.bfloat16),
        jax.ShapeDtypeStruct((N_V_PAD, D), jnp.bfloat16),
    ],
)

_stage2 = pl.pallas_call(
    _stage2_body,
    grid=(N_HE_PAD // BE,),
    in_specs=[
        pl.BlockSpec((BE, D), lambda i: (i, 0)),
        pl.BlockSpec((N_V_PAD, D), lambda i: (0, 0)),
        pl.BlockSpec((N_V_PAD, D), lambda i: (0, 0)),
        pl.BlockSpec((N_V_PAD, BE), lambda i: (0, i)),
        pl.BlockSpec((N_V_PAD, 8), lambda i: (0, 0)),
    ],
    out_specs=pl.BlockSpec((BE, D), lambda i: (i, 0)),
    out_shape=jax.ShapeDtypeStruct((N_HE_PAD, D), jnp.float32),
)


def kernel(vfeat, efeat, inc_node, inc_hedge, W_in, b_in, W1, b1, W2, b2,
           W3, b3, W4, b4, W5, b5, W6, b6):
    f32 = jnp.float32
    bf16 = jnp.bfloat16
    b_in2, b1_2, b2_2, b3_2, b4_2, b5_2, b6_2 = (
        b.reshape(1, D) for b in (b_in, b1, b2, b3, b4, b5, b6))

    rows = N_INC_P // D
    inc_node_p = jnp.pad(inc_node, (0, N_INC_P - N_INC),
                         constant_values=N_V_PAD)
    inc_hedge_p = jnp.pad(inc_hedge, (0, N_INC_P - N_INC),
                          constant_values=N_HE_PAD)
    s1 = _make_sidx()(inc_node_p.reshape(rows, D),
                      inc_hedge_p.reshape(rows, D))
    s1 = s1.reshape(N_CHUNKS, N_INC_P)

    zc = jnp.zeros((ZB,), f32)
    ones128 = jnp.ones((SC_CH,), f32)
    c1f = _count_build(s1, zc, ones128)
    c1 = c1f.reshape(N_V_PAD, N_HE_PAD)

    ef_p = jnp.pad(efeat, ((0, N_HE_PAD - N_HE), (0, 0)))
    vf_p = jnp.pad(vfeat, ((0, N_V_PAD - N_V), (0, 0)))

    kebf, vebf, qvbf, qebf = pl.pallas_call(
        _proj_body,
        out_shape=[
            jax.ShapeDtypeStruct((N_HE_PAD, D), bf16),
            jax.ShapeDtypeStruct((N_HE_PAD, D), bf16),
            jax.ShapeDtypeStruct((N_V_PAD, D), bf16),
            jax.ShapeDtypeStruct((N_HE_PAD, D), bf16),
        ],
    )(ef_p, vf_p, W_in, b_in2, W5, b5_2, W6, b6_2, W4, b4_2, W1, b1_2)

    ones8e = jnp.ones((N_HE_PAD, 8), bf16)
    ones8v = jnp.ones((N_V_PAD, 8), bf16)

    feat_v, kvbf, vvbf = _stage1(qvbf, kebf, vebf, c1, ones8e,
                                 W2, b2_2, W3, b3_2)
    feat_e2 = _stage2(qebf, kvbf, vvbf, c1, ones8v)

    return feat_v[:N_V], feat_e2[:N_HE]
